# Initial kernel scaffold; baseline (speedup 1.0000x reference)
import jax
import jax.numpy as jnp
from jax.experimental import pallas as pl


def kernel(source_pos, receiver_pos, absorption_coefficient, scattering_coefficient, radiance_pos, geometry, kernel_basis, delay_atten, envelope, row, col):
    raise NotImplementedError("write your pallas kernel here")


# speedup vs baseline: 80.0017x; 80.0017x over previous
import math

import jax
import jax.numpy as jnp
from jax import lax
from jax.experimental import pallas as pl
from jax.experimental.pallas import tpu as pltpu
from jax.experimental.pallas import tpu_sc as plsc

N = 10000
E = 320000
F = 128
NUM_BOUNCES = 8
FSM_GAMMA = 0.001
AIR = 0.001
FSM = math.exp(-math.log(FSM_GAMMA) / (NUM_BOUNCES * F))

NT = 16
EW = 20096
EP = NT * EW
NP = 10240
NSL = NP // NT


def _sqrt16(y):
    i = plsc.bitcast(y, jnp.int32)
    i = jnp.int32(0x1FBD1DF5) + lax.shift_right_logical(i, 1)
    x = plsc.bitcast(i, jnp.float32)
    for _ in range(3):
        x = 0.5 * (x + y / x)
    return x


def _sc_body(row_h, col_h, kb0_h, kb1_h, da_h, abs_h, sct_h,
             rx_h, ry_h, rz_h, gm_h, rcv_h, out_h,
             row_v, col_v, bufa_v, bufb_v, val_v, v_v, tab_v,
             acc_v, zero_v, rcv_v, px_v, py_v, pz_v, pg_v, u_sh):
    cid = lax.axis_index("c")
    wid = lax.axis_index("s")
    be = wid * EW
    bn = wid * NSL

    pltpu.sync_copy(row_h.at[pl.ds(be, EW)], row_v)
    pltpu.sync_copy(col_h.at[pl.ds(be, EW)], col_v)
    pltpu.sync_copy(kb0_h.at[pl.ds(be, EW)], bufa_v)
    pltpu.sync_copy(kb1_h.at[pl.ds(be, EW)], bufb_v)
    pltpu.sync_copy(da_h.at[pl.ds(be, EW)], val_v)
    pltpu.sync_copy(abs_h, v_v)
    pltpu.sync_copy(sct_h, tab_v)
    pltpu.sync_copy(rcv_h, rcv_v)

    def val_body(c, carry):
        o = c * 16
        cols = col_v[pl.ds(o, 16)]
        a = plsc.load_gather(v_v, [cols])
        s = plsc.load_gather(tab_v, [cols])
        kv = (1.0 - a) * (s * bufa_v[pl.ds(o, 16)]
                          + (1.0 - s) * bufb_v[pl.ds(o, 16)])
        val_v[pl.ds(o, 16)] = kv * val_v[pl.ds(o, 16)] * FSM
        return carry
    lax.fori_loop(0, EW // 16, val_body, 0)

    def z_body(k, carry):
        zero_v[pl.ds(k * 16, 16)] = jnp.zeros((16,), jnp.float32)
        return carry
    lax.fori_loop(0, NSL // 16, z_body, 0)

    pltpu.sync_copy(rx_h.at[pl.ds(bn, NSL)], px_v)
    pltpu.sync_copy(ry_h.at[pl.ds(bn, NSL)], py_v)
    pltpu.sync_copy(rz_h.at[pl.ds(bn, NSL)], pz_v)
    pltpu.sync_copy(gm_h.at[pl.ds(bn, NSL)], pg_v)

    def w_body(k, carry):
        o = k * 16
        dx = px_v[pl.ds(o, 16)] - rcv_v[pl.ds(0, 16)]
        dy = py_v[pl.ds(o, 16)] - rcv_v[pl.ds(16, 16)]
        dz = pz_v[pl.ds(o, 16)] - rcv_v[pl.ds(32, 16)]
        d2 = dx * dx + dy * dy + dz * dz
        d = _sqrt16(d2)
        acc_v[pl.ds(o, 16)] = (pg_v[pl.ds(o, 16)] / (d2 + 1.0)
                               * jnp.exp(-AIR * d))
        return carry
    lax.fori_loop(0, NSL // 16, w_body, 0)

    pltpu.sync_copy(acc_v, u_sh.at[pl.ds(bn, NSL)])
    plsc.subcore_barrier()
    pltpu.sync_copy(u_sh, v_v)
    plsc.subcore_barrier()

    for _ in range(NUM_BOUNCES):
        pltpu.sync_copy(zero_v, u_sh.at[pl.ds(bn, NSL)])
        plsc.subcore_barrier()

        def m_body(c, carry):
            o = c * 16
            rows = row_v[pl.ds(o, 16)]
            g = plsc.load_gather(v_v, [rows])
            bufa_v[pl.ds(o, 16)] = val_v[pl.ds(o, 16)] * g
            return carry
        lax.fori_loop(0, EW // 16, m_body, 0)

        pltpu.sync_copy(bufa_v, u_sh.at[col_v], add=True)
        plsc.subcore_barrier()

        pltpu.sync_copy(u_sh, v_v)

        def a_body(k, carry):
            o = k * 16
            acc_v[pl.ds(o, 16)] = (acc_v[pl.ds(o, 16)]
                                   + v_v[pl.ds(bn + o, 16)])
            return carry
        lax.fori_loop(0, NSL // 16, a_body, 0)
        plsc.subcore_barrier()

    @pl.when(cid == 0)
    def _():
        pltpu.sync_copy(acc_v, out_h.at[pl.ds(bn, NSL)])


def _sc_transfer(rowp, colp, kb0, kb1, dap, absp, sctp, rx, ry, rz, gmp, rcv):
    f32 = jnp.float32
    mesh = plsc.VectorSubcoreMesh(core_axis_name="c", subcore_axis_name="s",
                                  num_cores=2, num_subcores=NT)
    return pl.kernel(
        _sc_body,
        out_type=jax.ShapeDtypeStruct((NP,), f32),
        mesh=mesh,
        compiler_params=pltpu.CompilerParams(needs_layout_passes=False),
        scratch_types=[
            pltpu.VMEM((EW,), jnp.int32),
            pltpu.VMEM((EW,), jnp.int32),
            pltpu.VMEM((EW,), f32),
            pltpu.VMEM((EW,), f32),
            pltpu.VMEM((EW,), f32),
            pltpu.VMEM((NP,), f32),
            pltpu.VMEM((NP,), f32),
            pltpu.VMEM((NSL,), f32),
            pltpu.VMEM((NSL,), f32),
            pltpu.VMEM((48,), f32),
            pltpu.VMEM((NSL,), f32),
            pltpu.VMEM((NSL,), f32),
            pltpu.VMEM((NSL,), f32),
            pltpu.VMEM((NSL,), f32),
            pltpu.VMEM_SHARED((NP,), f32),
        ],
    )(rowp, colp, kb0, kb1, dap, absp, sctp, rx, ry, rz, gmp, rcv)


_LANES = 1024
_GRID = NP // _LANES


def _epi_body(acc_ref, rx_ref, ry_ref, rz_ref, gm_ref,
              src_ref, rcv_ref, env_ref, out_ref):
    j = pl.program_id(0)
    sx = src_ref[0, 0]
    sy = src_ref[0, 1]
    sz = src_ref[0, 2]
    dx = rx_ref[...] - sx
    dy = ry_ref[...] - sy
    dz = rz_ref[...] - sz
    d2 = dx * dx + dy * dy + dz * dz
    d = jnp.sqrt(d2)
    h = acc_ref[...] * gm_ref[...] / (d2 + 1.0)
    f1 = 1.0 + lax.broadcasted_iota(jnp.int32, (F, 1), 0).astype(jnp.float32)
    x = h * jnp.exp((-AIR) * d * f1)
    part = jnp.sum(x, axis=1, keepdims=True)

    @pl.when(j == 0)
    def _():
        out_ref[...] = part

    @pl.when(j > 0)
    def _():
        out_ref[...] = out_ref[...] + part

    @pl.when(j == _GRID - 1)
    def _():
        qx = rcv_ref[0, 0] - sx
        qy = rcv_ref[0, 1] - sy
        qz = rcv_ref[0, 2] - sz
        dd2 = qx * qx + qy * qy + qz * qz
        direct = jnp.exp(-AIR * jnp.sqrt(dd2)) / (dd2 + 1.0)
        out_ref[...] = (out_ref[...] + direct) * jnp.exp(env_ref[...])


def _epilogue(acc, rx, ry, rz, gm, src, rcv, env):
    f32 = jnp.float32
    vspec = pl.BlockSpec((1, _LANES), lambda j: (0, j))
    sspec = pl.BlockSpec(memory_space=pltpu.SMEM)
    return pl.pallas_call(
        _epi_body,
        grid=(_GRID,),
        in_specs=[vspec, vspec, vspec, vspec, vspec, sspec, sspec,
                  pl.BlockSpec((F, 1), lambda j: (0, 0))],
        out_specs=pl.BlockSpec((F, 1), lambda j: (0, 0)),
        out_shape=jax.ShapeDtypeStruct((F, 1), f32),
    )(acc, rx, ry, rz, gm, src, rcv, env)


def kernel(source_pos, receiver_pos, absorption_coefficient,
           scattering_coefficient, radiance_pos, geometry, kernel_basis,
           delay_atten, envelope, row, col):
    f32 = jnp.float32
    ezero = jnp.zeros((EP - E,), f32)
    rowp = jnp.concatenate([row, jnp.zeros((EP - E,), jnp.int32)])
    colp = jnp.concatenate([col, jnp.zeros((EP - E,), jnp.int32)])
    kb0 = jnp.concatenate([kernel_basis[:, 0], ezero])
    kb1 = jnp.concatenate([kernel_basis[:, 1], ezero])
    dap = jnp.concatenate([delay_atten, ezero])
    absp = jnp.pad(absorption_coefficient, (0, NP - N))
    sctp = jnp.pad(scattering_coefficient, (0, NP - N))
    rx = jnp.pad(radiance_pos[:, 0], (0, NP - N))
    ry = jnp.pad(radiance_pos[:, 1], (0, NP - N))
    rz = jnp.pad(radiance_pos[:, 2], (0, NP - N))
    gmp = jnp.pad(geometry, (0, NP - N))
    rcv = jnp.repeat(receiver_pos, 16)

    acc = _sc_transfer(rowp, colp, kb0, kb1, dap, absp, sctp,
                       rx, ry, rz, gmp, rcv)

    echo = _epilogue(acc.reshape(1, NP), rx.reshape(1, NP),
                     ry.reshape(1, NP), rz.reshape(1, NP),
                     gmp.reshape(1, NP), source_pos.reshape(1, 3),
                     receiver_pos.reshape(1, 3), envelope.reshape(F, 1))
    return echo.reshape(F)
